# baseline (speedup 1.0000x reference)
import dataclasses
import functools

import jax
import jax.numpy as jnp
from jax import lax
from jax.experimental import pallas as pl
from jax.experimental.pallas import tpu as pltpu
from jax.experimental.pallas import tpu_sc as plsc

N = 10000
D = 128
NC = 2
NS = 16
L = 16
NW = NC * NS
CH = 128
NCHUNK = 84
EPW = NCHUNK * CH
E_PAD = EPW * NW
CPP = NCHUNK // 2
PAIRS = CPP // 2
N_PAD = 10240
RPS = N // NS
NP_PS = N_PAD // NS

_mesh = plsc.VectorSubcoreMesh(core_axis_name="c", subcore_axis_name="s")

_sc_params = pltpu.CompilerParams()
if "needs_layout_passes" in pltpu.CompilerParams.__dataclass_fields__:
    _sc_params = dataclasses.replace(_sc_params, needs_layout_passes=False)


@functools.partial(
    pl.kernel, mesh=_mesh,
    out_type=jax.ShapeDtypeStruct((NC, N_PAD, D), jnp.float32),
    compiler_params=_sc_params,
    scratch_types=[
        pltpu.VMEM_SHARED((N_PAD, D), jnp.float32),
        pltpu.VMEM((CH, D), jnp.float32),
        pltpu.VMEM((1, CH), jnp.int32),
        pltpu.VMEM((1, CH), jnp.float32),
    ],
)
def _deg_kernel(dst_hbm, ew_hbm, out_hbm, acc, stg, dstb, ewb):
    c = lax.axis_index("c")
    s = lax.axis_index("s")

    @pl.loop(0, CH)
    def _(i):
        for q in range(D // L):
            stg[i, pl.ds(q * L, L)] = jnp.zeros((L,), jnp.float32)

    @pl.loop(0, NP_PS // CH)
    def _(t):
        pltpu.sync_copy(stg, acc.at[pl.ds(s * NP_PS + t * CH, CH), :])

    plsc.subcore_barrier()

    base = (c * NS + s) * EPW
    rows = lax.iota(jnp.int32, L)
    zcol = jnp.zeros((L,), jnp.int32)

    @pl.loop(0, NCHUNK)
    def _(k):
        off = base + k * CH
        pltpu.sync_copy(dst_hbm.at[pl.ds(off, CH)], dstb.at[0])
        pltpu.sync_copy(ew_hbm.at[pl.ds(off, CH)], ewb.at[0])
        for g in range(CH // L):
            ewv = ewb[0, pl.ds(g * L, L)]
            plsc.store_scatter(stg, [rows + g * L, zcol], ewv)
        pltpu.sync_copy(stg, acc.at[dstb.at[0]], add=True)

    plsc.subcore_barrier()

    @pl.loop(0, NP_PS // CH)
    def _(t):
        r0 = s * NP_PS + t * CH
        pltpu.sync_copy(acc.at[pl.ds(r0, CH), :],
                        out_hbm.at[c, pl.ds(r0, CH), :])


@functools.partial(
    pl.kernel, mesh=_mesh,
    out_type=jax.ShapeDtypeStruct((NC, N, D), jnp.float32),
    compiler_params=_sc_params,
    scratch_types=[
        pltpu.VMEM_SHARED((N, D), jnp.float32),
        pltpu.VMEM((CH, D), jnp.float32),
        pltpu.VMEM((CH, D), jnp.float32),
        pltpu.VMEM((CPP, 1, CH), jnp.int32),
        pltpu.VMEM((CPP, 1, CH), jnp.int32),
        pltpu.VMEM((CPP, 1, CH), jnp.float32),
        pltpu.SemaphoreType.DMA,
        pltpu.SemaphoreType.DMA,
        pltpu.SemaphoreType.DMA,
        pltpu.SemaphoreType.DMA,
    ],
)
def _agg_kernel(g_hbm, src_hbm, dst_hbm, ew_hbm, out_hbm,
                acc, rowb0, rowb1, srcw, dstw, eww,
                sg0, sg1, ss0, ss1):
    c = lax.axis_index("c")
    s = lax.axis_index("s")

    @pl.loop(0, CH)
    def _(i):
        for q in range(D // L):
            rowb0[i, pl.ds(q * L, L)] = jnp.zeros((L,), jnp.float32)

    @pl.loop(0, 8)
    def _(t):
        r0 = s * 640 + t * 80

        @pl.when(r0 < N)
        def _():
            pltpu.sync_copy(rowb0.at[pl.ds(0, 80), :],
                            acc.at[pl.ds(r0, 80), :])

    w = c * NS + s

    def compute(k, rowb):
        @pl.loop(0, CH // L)
        def _(gi):
            nvec = eww[k, 0, pl.ds(gi * L, L)]
            for lane in range(L):
                nb = lax.broadcast_in_dim(nvec[lane], (L,), ())
                for q in range(D // L):
                    e = gi * L + lane
                    rowb[e, pl.ds(q * L, L)] = rowb[e, pl.ds(q * L, L)] * nb

    first = True
    for ph in range(NCHUNK // CPP):
        base = w * NCHUNK + ph * CPP
        pltpu.sync_copy(src_hbm.at[pl.ds(base, CPP), :, :], srcw)
        pltpu.sync_copy(dst_hbm.at[pl.ds(base, CPP), :, :], dstw)
        pltpu.sync_copy(ew_hbm.at[pl.ds(base, CPP), :, :], eww)
        if first:
            plsc.subcore_barrier()
            first = False

        pltpu.async_copy(g_hbm.at[srcw.at[0, 0]], rowb0, sg0)

        @pl.loop(0, PAIRS)
        def _(t):
            ka = 2 * t
            kb = 2 * t + 1

            @pl.when(t > 0)
            def _():
                pltpu.make_async_copy(rowb1, acc.at[dstw.at[kb - 2, 0]],
                                      ss1).wait()

            pltpu.async_copy(g_hbm.at[srcw.at[kb, 0]], rowb1, sg1)
            pltpu.make_async_copy(g_hbm.at[srcw.at[ka, 0]], rowb0, sg0).wait()
            compute(ka, rowb0)
            pltpu.async_copy(rowb0, acc.at[dstw.at[ka, 0]], ss0, add=True)
            pltpu.make_async_copy(g_hbm.at[srcw.at[kb, 0]], rowb1, sg1).wait()
            pltpu.make_async_copy(rowb0, acc.at[dstw.at[ka, 0]], ss0).wait()

            @pl.when(t < PAIRS - 1)
            def _():
                pltpu.async_copy(g_hbm.at[srcw.at[ka + 2, 0]], rowb0, sg0)

            compute(kb, rowb1)
            pltpu.async_copy(rowb1, acc.at[dstw.at[kb, 0]], ss1, add=True)

        pltpu.make_async_copy(rowb1, acc.at[dstw.at[CPP - 1, 0]], ss1).wait()

    plsc.subcore_barrier()

    @pl.loop(0, 8)
    def _(t):
        r0 = s * 640 + t * 80

        @pl.when(r0 < N)
        def _():
            pltpu.sync_copy(acc.at[pl.ds(r0, 80), :],
                            out_hbm.at[c, pl.ds(r0, 80), :])


def _dinv_kernel(degp):
    def body(d_ref, o_ref):
        dsum = d_ref[0] + d_ref[1]
        o_ref[...] = jnp.where(
            dsum > 0, lax.rsqrt(jnp.maximum(dsum, 1e-12)), 0.0)

    return pl.pallas_call(
        body,
        grid=(1,),
        in_specs=[pl.BlockSpec((2, 80, 128), lambda i: (0, 0, 0))],
        out_specs=pl.BlockSpec((80, 128), lambda i: (0, 0)),
        out_shape=jax.ShapeDtypeStruct((80, 128), jnp.float32),
    )(degp)


def _mm1(x, W, dcol):
    def body(x_ref, w_ref, d_ref, o_ref):
        o_ref[...] = d_ref[...] * jnp.dot(x_ref[...], w_ref[...],
                                          preferred_element_type=jnp.float32)

    return pl.pallas_call(
        body,
        grid=(10,),
        in_specs=[pl.BlockSpec((1000, D), lambda i: (i, 0)),
                  pl.BlockSpec((D, D), lambda i: (0, 0)),
                  pl.BlockSpec((1000, 1), lambda i: (i, 0))],
        out_specs=pl.BlockSpec((1000, D), lambda i: (i, 0)),
        out_shape=jax.ShapeDtypeStruct((N, D), jnp.float32),
    )(x, W, dcol)


def _mm_fused(p, b, W, dcol):
    def body(p0_ref, p1_ref, b_ref, w_ref, d_ref, o_ref):
        h = jnp.maximum(d_ref[...] * (p0_ref[0] + p1_ref[0]) + b_ref[...],
                        0.0)
        o_ref[...] = d_ref[...] * jnp.dot(h, w_ref[...],
                                          preferred_element_type=jnp.float32)

    return pl.pallas_call(
        body,
        grid=(10,),
        in_specs=[pl.BlockSpec((1, 1000, D), lambda i: (0, i, 0)),
                  pl.BlockSpec((1, 1000, D), lambda i: (1, i, 0)),
                  pl.BlockSpec((1, D), lambda i: (0, 0)),
                  pl.BlockSpec((D, D), lambda i: (0, 0)),
                  pl.BlockSpec((1000, 1), lambda i: (i, 0))],
        out_specs=pl.BlockSpec((1000, D), lambda i: (i, 0)),
        out_shape=jax.ShapeDtypeStruct((N, D), jnp.float32),
    )(p, p, b, W, dcol)


def _final(p, b, dcol):
    def body(p0_ref, p1_ref, b_ref, d_ref, o_ref):
        o_ref[...] = d_ref[...] * (p0_ref[0] + p1_ref[0]) + b_ref[...]

    return pl.pallas_call(
        body,
        grid=(10,),
        in_specs=[pl.BlockSpec((1, 1000, D), lambda i: (0, i, 0)),
                  pl.BlockSpec((1, 1000, D), lambda i: (1, i, 0)),
                  pl.BlockSpec((1, D), lambda i: (0, 0)),
                  pl.BlockSpec((1000, 1), lambda i: (i, 0))],
        out_specs=pl.BlockSpec((1000, D), lambda i: (i, 0)),
        out_shape=jax.ShapeDtypeStruct((N, D), jnp.float32),
    )(p, p, b, dcol)


def kernel(x, edge_index, edge_weight, W1, b1, W2, b2, W3, b3):
    loop_idx = jnp.arange(N, dtype=edge_index.dtype)
    src = jnp.concatenate([edge_index[0], loop_idx])
    dst = jnp.concatenate([edge_index[1], loop_idx])
    ew = jnp.concatenate([edge_weight, jnp.ones((N,), edge_weight.dtype)])
    pad = E_PAD - src.shape[0]
    src = jnp.pad(src, (0, pad))
    dst = jnp.pad(dst, (0, pad))
    ew = jnp.pad(ew, (0, pad))

    src3 = src.reshape(E_PAD // CH, 1, CH)
    dst3 = dst.reshape(E_PAD // CH, 1, CH)
    ew3 = ew.reshape(E_PAD // CH, 1, CH)

    degp = _deg_kernel(dst, ew)[:, :, 0].reshape(NC, 80, 128)
    dinv = _dinv_kernel(degp)
    dcol = dinv.reshape(N_PAD)[:N].reshape(N, 1)
    g = _mm1(x, W1, dcol)
    p = _agg_kernel(g, src3, dst3, ew3)
    g = _mm_fused(p, b1.reshape(1, D), W2, dcol)
    p = _agg_kernel(g, src3, dst3, ew3)
    g = _mm_fused(p, b2.reshape(1, D), W3, dcol)
    p = _agg_kernel(g, src3, dst3, ew3)
    return _final(p, b3.reshape(1, D), dcol)


# speedup vs baseline: 1.0562x; 1.0562x over previous
import dataclasses
import functools

import jax
import jax.numpy as jnp
from jax import lax
from jax.experimental import pallas as pl
from jax.experimental.pallas import tpu as pltpu
from jax.experimental.pallas import tpu_sc as plsc

N = 10000
D = 128
NC = 2
NS = 16
L = 16
NW = NC * NS
CH = 128
NCHUNK = 84
EPW = NCHUNK * CH
E_PAD = EPW * NW
CPP = NCHUNK // 2
PAIRS = CPP // 2
N_PAD = 10240
RPS = N // NS
NP_PS = N_PAD // NS

_mesh = plsc.VectorSubcoreMesh(core_axis_name="c", subcore_axis_name="s")

_sc_params = pltpu.CompilerParams()
if "needs_layout_passes" in pltpu.CompilerParams.__dataclass_fields__:
    _sc_params = dataclasses.replace(_sc_params, needs_layout_passes=False)


@functools.partial(
    pl.kernel, mesh=_mesh,
    out_type=jax.ShapeDtypeStruct((NC, N_PAD, D), jnp.float32),
    compiler_params=_sc_params,
    scratch_types=[
        pltpu.VMEM_SHARED((N_PAD, D), jnp.float32),
        pltpu.VMEM((CH, D), jnp.float32),
        pltpu.VMEM((1, CH), jnp.int32),
        pltpu.VMEM((1, CH), jnp.float32),
    ],
)
def _deg_kernel(dst_hbm, ew_hbm, out_hbm, acc, stg, dstb, ewb):
    c = lax.axis_index("c")
    s = lax.axis_index("s")

    @pl.loop(0, CH)
    def _(i):
        for q in range(D // L):
            stg[i, pl.ds(q * L, L)] = jnp.zeros((L,), jnp.float32)

    @pl.loop(0, NP_PS // CH)
    def _(t):
        pltpu.sync_copy(stg, acc.at[pl.ds(s * NP_PS + t * CH, CH), :])

    plsc.subcore_barrier()

    base = (c * NS + s) * EPW
    rows = lax.iota(jnp.int32, L)
    zcol = jnp.zeros((L,), jnp.int32)

    @pl.loop(0, NCHUNK)
    def _(k):
        off = base + k * CH
        pltpu.sync_copy(dst_hbm.at[pl.ds(off, CH)], dstb.at[0])
        pltpu.sync_copy(ew_hbm.at[pl.ds(off, CH)], ewb.at[0])
        for g in range(CH // L):
            ewv = ewb[0, pl.ds(g * L, L)]
            plsc.store_scatter(stg, [rows + g * L, zcol], ewv)
        pltpu.sync_copy(stg, acc.at[dstb.at[0]], add=True)

    plsc.subcore_barrier()

    @pl.loop(0, NP_PS // CH)
    def _(t):
        r0 = s * NP_PS + t * CH
        pltpu.sync_copy(acc.at[pl.ds(r0, CH), :],
                        out_hbm.at[c, pl.ds(r0, CH), :])


@functools.partial(
    pl.kernel, mesh=_mesh,
    out_type=jax.ShapeDtypeStruct((NC, N, D), jnp.float32),
    compiler_params=_sc_params,
    scratch_types=[
        pltpu.VMEM_SHARED((N, D), jnp.float32),
        pltpu.VMEM((CH, D), jnp.float32),
        pltpu.VMEM((CH, D), jnp.float32),
        pltpu.VMEM((CPP, 1, CH), jnp.int32),
        pltpu.VMEM((CPP, 1, CH), jnp.int32),
        pltpu.VMEM((CPP, 1, CH), jnp.float32),
        pltpu.SemaphoreType.DMA,
        pltpu.SemaphoreType.DMA,
        pltpu.SemaphoreType.DMA,
        pltpu.SemaphoreType.DMA,
    ],
)
def _agg_kernel(g_hbm, src_hbm, dst_hbm, ew_hbm, out_hbm,
                acc, rowb0, rowb1, srcw, dstw, eww,
                sg0, sg1, ss0, ss1):
    c = lax.axis_index("c")
    s = lax.axis_index("s")

    @pl.loop(0, CH)
    def _(i):
        for q in range(D // L):
            rowb0[i, pl.ds(q * L, L)] = jnp.zeros((L,), jnp.float32)

    @pl.loop(0, 8)
    def _(t):
        r0 = s * 640 + t * 80

        @pl.when(r0 < N)
        def _():
            pltpu.sync_copy(rowb0.at[pl.ds(0, 80), :],
                            acc.at[pl.ds(r0, 80), :])

    w = c * NS + s

    def compute(k, rowb):
        @pl.loop(0, CH // L)
        def _(gi):
            nvec = eww[k, 0, pl.ds(gi * L, L)]
            for lane in range(L):
                nb = lax.broadcast_in_dim(nvec[lane], (L,), ())
                for q in range(D // L):
                    e = gi * L + lane
                    rowb[e, pl.ds(q * L, L)] = rowb[e, pl.ds(q * L, L)] * nb

    first = True
    for ph in range(NCHUNK // CPP):
        base = w * NCHUNK + ph * CPP
        pltpu.sync_copy(src_hbm.at[pl.ds(base, CPP), :, :], srcw)
        pltpu.sync_copy(dst_hbm.at[pl.ds(base, CPP), :, :], dstw)
        pltpu.sync_copy(ew_hbm.at[pl.ds(base, CPP), :, :], eww)
        if first:
            plsc.subcore_barrier()
            first = False

        pltpu.async_copy(g_hbm.at[srcw.at[0, 0]], rowb0, sg0)

        @pl.loop(0, PAIRS)
        def _(t):
            ka = 2 * t
            kb = 2 * t + 1

            pltpu.async_copy(g_hbm.at[srcw.at[kb, 0]], rowb1, sg1)
            pltpu.make_async_copy(g_hbm.at[srcw.at[ka, 0]], rowb0, sg0).wait()
            compute(ka, rowb0)
            pltpu.sync_copy(rowb0, acc.at[dstw.at[ka, 0]], add=True)

            @pl.when(t < PAIRS - 1)
            def _():
                pltpu.async_copy(g_hbm.at[srcw.at[ka + 2, 0]], rowb0, sg0)

            pltpu.make_async_copy(g_hbm.at[srcw.at[kb, 0]], rowb1, sg1).wait()
            compute(kb, rowb1)
            pltpu.sync_copy(rowb1, acc.at[dstw.at[kb, 0]], add=True)

    plsc.subcore_barrier()

    @pl.loop(0, 8)
    def _(t):
        r0 = s * 640 + t * 80

        @pl.when(r0 < N)
        def _():
            pltpu.sync_copy(acc.at[pl.ds(r0, 80), :],
                            out_hbm.at[c, pl.ds(r0, 80), :])


def _dinv_kernel(degp):
    def body(d_ref, o_ref):
        dsum = d_ref[0] + d_ref[1]
        o_ref[...] = jnp.where(
            dsum > 0, lax.rsqrt(jnp.maximum(dsum, 1e-12)), 0.0)

    return pl.pallas_call(
        body,
        grid=(1,),
        in_specs=[pl.BlockSpec((2, 80, 128), lambda i: (0, 0, 0))],
        out_specs=pl.BlockSpec((80, 128), lambda i: (0, 0)),
        out_shape=jax.ShapeDtypeStruct((80, 128), jnp.float32),
    )(degp)


def _mm1(x, W, dcol):
    def body(x_ref, w_ref, d_ref, o_ref):
        o_ref[...] = d_ref[...] * jnp.dot(x_ref[...], w_ref[...],
                                          preferred_element_type=jnp.float32)

    return pl.pallas_call(
        body,
        grid=(10,),
        in_specs=[pl.BlockSpec((1000, D), lambda i: (i, 0)),
                  pl.BlockSpec((D, D), lambda i: (0, 0)),
                  pl.BlockSpec((1000, 1), lambda i: (i, 0))],
        out_specs=pl.BlockSpec((1000, D), lambda i: (i, 0)),
        out_shape=jax.ShapeDtypeStruct((N, D), jnp.float32),
    )(x, W, dcol)


def _mm_fused(p, b, W, dcol):
    def body(p0_ref, p1_ref, b_ref, w_ref, d_ref, o_ref):
        h = jnp.maximum(d_ref[...] * (p0_ref[0] + p1_ref[0]) + b_ref[...],
                        0.0)
        o_ref[...] = d_ref[...] * jnp.dot(h, w_ref[...],
                                          preferred_element_type=jnp.float32)

    return pl.pallas_call(
        body,
        grid=(10,),
        in_specs=[pl.BlockSpec((1, 1000, D), lambda i: (0, i, 0)),
                  pl.BlockSpec((1, 1000, D), lambda i: (1, i, 0)),
                  pl.BlockSpec((1, D), lambda i: (0, 0)),
                  pl.BlockSpec((D, D), lambda i: (0, 0)),
                  pl.BlockSpec((1000, 1), lambda i: (i, 0))],
        out_specs=pl.BlockSpec((1000, D), lambda i: (i, 0)),
        out_shape=jax.ShapeDtypeStruct((N, D), jnp.float32),
    )(p, p, b, W, dcol)


def _final(p, b, dcol):
    def body(p0_ref, p1_ref, b_ref, d_ref, o_ref):
        o_ref[...] = d_ref[...] * (p0_ref[0] + p1_ref[0]) + b_ref[...]

    return pl.pallas_call(
        body,
        grid=(10,),
        in_specs=[pl.BlockSpec((1, 1000, D), lambda i: (0, i, 0)),
                  pl.BlockSpec((1, 1000, D), lambda i: (1, i, 0)),
                  pl.BlockSpec((1, D), lambda i: (0, 0)),
                  pl.BlockSpec((1000, 1), lambda i: (i, 0))],
        out_specs=pl.BlockSpec((1000, D), lambda i: (i, 0)),
        out_shape=jax.ShapeDtypeStruct((N, D), jnp.float32),
    )(p, p, b, dcol)


def kernel(x, edge_index, edge_weight, W1, b1, W2, b2, W3, b3):
    loop_idx = jnp.arange(N, dtype=edge_index.dtype)
    src = jnp.concatenate([edge_index[0], loop_idx])
    dst = jnp.concatenate([edge_index[1], loop_idx])
    ew = jnp.concatenate([edge_weight, jnp.ones((N,), edge_weight.dtype)])
    pad = E_PAD - src.shape[0]
    src = jnp.pad(src, (0, pad))
    dst = jnp.pad(dst, (0, pad))
    ew = jnp.pad(ew, (0, pad))

    src3 = src.reshape(E_PAD // CH, 1, CH)
    dst3 = dst.reshape(E_PAD // CH, 1, CH)
    ew3 = ew.reshape(E_PAD // CH, 1, CH)

    degp = _deg_kernel(dst, ew)[:, :, 0].reshape(NC, 80, 128)
    dinv = _dinv_kernel(degp)
    dcol = dinv.reshape(N_PAD)[:N].reshape(N, 1)
    g = _mm1(x, W1, dcol)
    p = _agg_kernel(g, src3, dst3, ew3)
    g = _mm_fused(p, b1.reshape(1, D), W2, dcol)
    p = _agg_kernel(g, src3, dst3, ew3)
    g = _mm_fused(p, b2.reshape(1, D), W3, dcol)
    p = _agg_kernel(g, src3, dst3, ew3)
    return _final(p, b3.reshape(1, D), dcol)
